# speedup vs baseline: 1.0587x; 1.0587x over previous
import jax
import jax.numpy as jnp
from jax import lax
from jax.experimental import pallas as pl
from jax.experimental.pallas import tpu as pltpu
from jax.experimental.pallas import tpu_sc as plsc

_BATCH = 16384
_NC = 2
_NS = 16
_NW = _NC * _NS
_BPW = _BATCH // _NW
_LANES = 16


def _sc_body(users_hbm, items_hbm, ubias_hbm, ibias_hbm, out_hbm,
             uidx_v, iidx_v, urows_v, irows_v, sem_u, sem_i):
    wid = lax.axis_index("s") * _NC + lax.axis_index("c")
    base = wid * _BPW
    pltpu.sync_copy(users_hbm.at[pl.ds(base, _BPW)], uidx_v)
    pltpu.sync_copy(items_hbm.at[pl.ds(base, _BPW)], iidx_v)
    cu = pltpu.make_async_copy(ubias_hbm.at[uidx_v], urows_v, sem_u)
    ci = pltpu.make_async_copy(ibias_hbm.at[iidx_v], irows_v, sem_i)
    cu.start()
    ci.start()
    cu.wait()
    ci.wait()
    for j in range(_BPW // _LANES):
        sl = pl.ds(j * _LANES, _LANES)
        urows_v[sl] = urows_v[sl] + irows_v[sl]
    pltpu.sync_copy(urows_v, out_hbm.at[pl.ds(base, _BPW)])


@jax.jit
def kernel(users, items, user_bias, item_bias):
    mesh = plsc.VectorSubcoreMesh(core_axis_name="c", subcore_axis_name="s")
    k = pl.kernel(
        _sc_body,
        mesh=mesh,
        out_type=jax.ShapeDtypeStruct((_BATCH,), jnp.float32),
        scratch_types=[
            pltpu.VMEM((_BPW,), jnp.int32),
            pltpu.VMEM((_BPW,), jnp.int32),
            pltpu.VMEM((_BPW,), jnp.float32),
            pltpu.VMEM((_BPW,), jnp.float32),
            pltpu.SemaphoreType.DMA,
            pltpu.SemaphoreType.DMA,
        ],
    )
    return k(users.astype(jnp.int32), items.astype(jnp.int32),
             user_bias.reshape(-1), item_bias.reshape(-1))


# speedup vs baseline: 1.0638x; 1.0048x over previous
import jax
import jax.numpy as jnp
from jax import lax
from jax.experimental import pallas as pl
from jax.experimental.pallas import tpu as pltpu
from jax.experimental.pallas import tpu_sc as plsc

_BATCH = 16384
_NC = 2
_NS = 16
_NW = _NC * _NS
_BPW = _BATCH // _NW
_LANES = 16


def _sc_body(users_hbm, items_hbm, ubias_hbm, ibias_hbm, out_hbm,
             uidx_v, iidx_v, uflat_v, iflat_v,
             sem_ui, sem_ii, sem_u, sem_i):
    wid = lax.axis_index("s") * _NC + lax.axis_index("c")
    base = wid * _BPW
    cui = pltpu.make_async_copy(users_hbm.at[pl.ds(base, _BPW)], uidx_v, sem_ui)
    cii = pltpu.make_async_copy(items_hbm.at[pl.ds(base, _BPW)], iidx_v, sem_ii)
    cui.start()
    cii.start()
    cui.wait()
    cu = pltpu.make_async_copy(ubias_hbm.at[uidx_v], uflat_v, sem_u)
    cu.start()
    cii.wait()
    ci = pltpu.make_async_copy(ibias_hbm.at[iidx_v], iflat_v, sem_i)
    ci.start()
    cu.wait()
    ci.wait()
    for j in range(_BPW // _LANES):
        sl = pl.ds(j * _LANES, _LANES)
        uflat_v[sl] = uflat_v[sl] + iflat_v[sl]
    pltpu.sync_copy(uflat_v, out_hbm.at[pl.ds(base, _BPW)])


@jax.jit
def kernel(users, items, user_bias, item_bias):
    mesh = plsc.VectorSubcoreMesh(core_axis_name="c", subcore_axis_name="s")
    k = pl.kernel(
        _sc_body,
        mesh=mesh,
        out_type=jax.ShapeDtypeStruct((_BATCH,), jnp.float32),
        scratch_types=[
            pltpu.VMEM((_BPW,), jnp.int32),
            pltpu.VMEM((_BPW,), jnp.int32),
            pltpu.VMEM((_BPW,), jnp.float32),
            pltpu.VMEM((_BPW,), jnp.float32),
            pltpu.SemaphoreType.DMA,
            pltpu.SemaphoreType.DMA,
            pltpu.SemaphoreType.DMA,
            pltpu.SemaphoreType.DMA,
        ],
    )
    return k(users.astype(jnp.int32), items.astype(jnp.int32),
             user_bias.reshape(-1), item_bias.reshape(-1))
